# Initial kernel scaffold; baseline (speedup 1.0000x reference)
import jax
import jax.numpy as jnp
from jax.experimental import pallas as pl


def kernel(x, edge_index):
    raise NotImplementedError("write your pallas kernel here")


# speedup vs baseline: 10.1482x; 10.1482x over previous
import functools

import jax
import jax.numpy as jnp
from jax import lax
from jax.experimental import pallas as pl
from jax.experimental.pallas import tpu as pltpu
from jax.experimental.pallas import tpu_sc as plsc

N = 10000
E = 320000
D = 128

NC = 2
NS = 16
NW = NC * NS
EPW = E // NW
BATCH = 125
CHUNKS = EPW // BATCH
NP = 10240
RPT = NP // NS

_SC_PARAMS = pltpu.CompilerParams(needs_layout_passes=False)


def _sc_counts(dst_flat):
  mesh = plsc.VectorSubcoreMesh(core_axis_name="c", subcore_axis_name="s")

  @functools.partial(
      pl.kernel,
      out_type=jax.ShapeDtypeStruct((NW, 1, N), jnp.float32),
      mesh=mesh,
      compiler_params=_SC_PARAMS,
      scratch_types=[
          pltpu.VMEM((EPW,), jnp.int32),
          pltpu.VMEM((N,), jnp.float32),
      ],
  )
  def body(dstf_hbm, cnt_hbm, dstf_v, cnt_v):
    c = lax.axis_index("c").astype(jnp.int32)
    s = lax.axis_index("s").astype(jnp.int32)
    wid = s * jnp.int32(NC) + c

    pltpu.sync_copy(dstf_hbm.at[wid, jnp.int32(0)], dstf_v)

    zeros16 = jnp.zeros((16,), jnp.float32)
    ones16 = jnp.ones((16,), jnp.float32)

    @pl.loop(jnp.int32(0), jnp.int32(N // 16))
    def _zero(i):
      cnt_v[pl.ds(i * jnp.int32(16), 16)] = zeros16

    @pl.loop(jnp.int32(0), jnp.int32(EPW // 16))
    def _count(i):
      idx = dstf_v[pl.ds(i * jnp.int32(16), 16)]
      plsc.addupdate_scatter(cnt_v, [idx], ones16)

    pltpu.sync_copy(cnt_v, cnt_hbm.at[wid, jnp.int32(0)])

  return body(dst_flat)


def _sc_accumulate(x, src_idx, dst_idx, zeros):
  mesh = plsc.VectorSubcoreMesh(core_axis_name="c", subcore_axis_name="s")

  @functools.partial(
      pl.kernel,
      out_type=jax.ShapeDtypeStruct((NC, NP, D), jnp.float32),
      mesh=mesh,
      compiler_params=_SC_PARAMS,
      scratch_types=[
          pltpu.VMEM((CHUNKS, BATCH), jnp.int32),
          pltpu.VMEM((CHUNKS, BATCH), jnp.int32),
          pltpu.VMEM((BATCH, D), jnp.float32),
          pltpu.VMEM_SHARED((NP, D), jnp.float32),
      ],
  )
  def body(x_hbm, srci_hbm, dsti_hbm, zeros_hbm, part_hbm,
           srci_v, dsti_v, rows_v, acc_sh):
    c = lax.axis_index("c").astype(jnp.int32)
    s = lax.axis_index("s").astype(jnp.int32)
    wid = s * jnp.int32(NC) + c

    pltpu.sync_copy(srci_hbm.at[wid], srci_v)
    pltpu.sync_copy(dsti_hbm.at[wid], dsti_v)

    row0 = pl.multiple_of(s * jnp.int32(RPT), RPT)
    pltpu.sync_copy(zeros_hbm.at[pl.ds(row0, RPT)],
                    acc_sh.at[pl.ds(row0, RPT)])

    plsc.subcore_barrier()

    @pl.loop(jnp.int32(0), jnp.int32(CHUNKS))
    def _chunk(g):
      pltpu.sync_copy(x_hbm.at[srci_v.at[g]], rows_v)
      pltpu.sync_copy(rows_v, acc_sh.at[dsti_v.at[g]], add=True)

    plsc.subcore_barrier()

    pltpu.sync_copy(acc_sh.at[pl.ds(row0, RPT)],
                    part_hbm.at[c].at[pl.ds(row0, RPT)])

  return body(x, src_idx, dst_idx, zeros)


def _tc_combine(partials, counts):
  def body(p_ref, c_ref, o_ref):
    summed = p_ref[0, :N, :] + p_ref[1, :N, :]
    cnt = jnp.sum(c_ref[:, 0, :], axis=0)
    cnt = jnp.maximum(cnt, 1.0)
    o_ref[...] = summed / cnt[:, None]

  return pl.pallas_call(
      body,
      out_shape=jax.ShapeDtypeStruct((N, D), jnp.float32),
  )(partials, counts)


@jax.jit
def kernel(x, edge_index):
  src = edge_index[0].astype(jnp.int32).reshape(NW, CHUNKS, BATCH)
  dst = edge_index[1].astype(jnp.int32)
  dst_rows = dst.reshape(NW, CHUNKS, BATCH)
  dst_flat = dst.reshape(NW, 1, EPW)
  zeros = jnp.zeros((NP, D), jnp.float32)
  x = x.astype(jnp.float32)
  counts = _sc_counts(dst_flat)
  partials = _sc_accumulate(x, src, dst_rows, zeros)
  return _tc_combine(partials, counts)


# speedup vs baseline: 14.1087x; 1.3903x over previous
import functools

import jax
import jax.numpy as jnp
from jax import lax
from jax.experimental import pallas as pl
from jax.experimental.pallas import tpu as pltpu
from jax.experimental.pallas import tpu_sc as plsc

N = 10000
E = 320000
D = 128

NC = 2
NS = 16
NW = NC * NS
EPW = E // NW
BATCH = 125
CHUNKS = EPW // BATCH
NP = 10240
RPT = NP // NS

_SC_PARAMS = pltpu.CompilerParams(needs_layout_passes=False)


def _sc_counts(dst_flat):
  mesh = plsc.VectorSubcoreMesh(core_axis_name="c", subcore_axis_name="s")

  @functools.partial(
      pl.kernel,
      out_type=jax.ShapeDtypeStruct((NW, 1, N), jnp.float32),
      mesh=mesh,
      compiler_params=_SC_PARAMS,
      scratch_types=[
          pltpu.VMEM((EPW,), jnp.int32),
          pltpu.VMEM((N,), jnp.float32),
      ],
  )
  def body(dstf_hbm, cnt_hbm, dstf_v, cnt_v):
    c = lax.axis_index("c").astype(jnp.int32)
    s = lax.axis_index("s").astype(jnp.int32)
    wid = s * jnp.int32(NC) + c

    pltpu.sync_copy(dstf_hbm.at[wid, jnp.int32(0)], dstf_v)

    zeros16 = jnp.zeros((16,), jnp.float32)
    ones16 = jnp.ones((16,), jnp.float32)

    @pl.loop(jnp.int32(0), jnp.int32(N // 16))
    def _zero(i):
      cnt_v[pl.ds(i * jnp.int32(16), 16)] = zeros16

    @pl.loop(jnp.int32(0), jnp.int32(EPW // 16))
    def _count(i):
      idx = dstf_v[pl.ds(i * jnp.int32(16), 16)]
      plsc.addupdate_scatter(cnt_v, [idx], ones16)

    pltpu.sync_copy(cnt_v, cnt_hbm.at[wid, jnp.int32(0)])

  return body(dst_flat)


HC = CHUNKS // 2


def _sc_accumulate(x, src_idx, dst_idx, zeros):
  mesh = plsc.VectorSubcoreMesh(core_axis_name="c", subcore_axis_name="s")

  @functools.partial(
      pl.kernel,
      out_type=jax.ShapeDtypeStruct((NC, NP, D), jnp.float32),
      mesh=mesh,
      compiler_params=_SC_PARAMS,
      scratch_types=[
          pltpu.VMEM((HC, BATCH), jnp.int32),
          pltpu.VMEM((HC, BATCH), jnp.int32),
          pltpu.VMEM((2, BATCH, D), jnp.float32),
          pltpu.VMEM_SHARED((NP, D), jnp.float32),
          pltpu.SemaphoreType.DMA((2,)),
          pltpu.SemaphoreType.DMA((2,)),
      ],
  )
  def body(x_hbm, srci_hbm, dsti_hbm, zeros_hbm, part_hbm,
           srci_v, dsti_v, rows_v, acc_sh, gsem, ssem):
    c = lax.axis_index("c").astype(jnp.int32)
    s = lax.axis_index("s").astype(jnp.int32)
    wid = s * jnp.int32(NC) + c

    row0 = pl.multiple_of(s * jnp.int32(RPT), RPT)
    pltpu.sync_copy(zeros_hbm.at[pl.ds(row0, RPT)],
                    acc_sh.at[pl.ds(row0, RPT)])

    plsc.subcore_barrier()

    def gather_start(g, b):
      pltpu.async_copy(x_hbm.at[srci_v.at[g]], rows_v.at[b], gsem.at[b])

    def gather_wait(g, b):
      pltpu.make_async_copy(
          x_hbm.at[srci_v.at[g]], rows_v.at[b], gsem.at[b]).wait()

    def scatter_start(g, b):
      pltpu.async_copy(rows_v.at[b], acc_sh.at[dsti_v.at[g]], ssem.at[b],
                       add=True)

    def scatter_wait(g, b):
      pltpu.make_async_copy(
          rows_v.at[b], acc_sh.at[dsti_v.at[g]], ssem.at[b]).wait()

    for st in range(2):
      pltpu.sync_copy(srci_hbm.at[wid, pl.ds(st * HC, HC)], srci_v)
      pltpu.sync_copy(dsti_hbm.at[wid, pl.ds(st * HC, HC)], dsti_v)

      gather_start(jnp.int32(0), jnp.int32(0))

      @pl.loop(jnp.int32(0), jnp.int32(HC))
      def _chunk(g):
        b = jnp.bitwise_and(g, jnp.int32(1))
        nb = jnp.int32(1) - b

        @pl.when(g >= jnp.int32(1))
        def _():
          scatter_wait(g - jnp.int32(1), nb)

        @pl.when(g + jnp.int32(1) < jnp.int32(HC))
        def _():
          gather_start(g + jnp.int32(1), nb)

        gather_wait(g, b)
        scatter_start(g, b)

      scatter_wait(jnp.int32(HC - 1), jnp.int32((HC - 1) & 1))

    plsc.subcore_barrier()

    pltpu.sync_copy(acc_sh.at[pl.ds(row0, RPT)],
                    part_hbm.at[c].at[pl.ds(row0, RPT)])

  return body(x, src_idx, dst_idx, zeros)


def _tc_combine(partials, counts):
  def body(p_ref, c_ref, o_ref):
    summed = p_ref[0, :N, :] + p_ref[1, :N, :]
    cnt = jnp.sum(c_ref[:, 0, :], axis=0)
    cnt = jnp.maximum(cnt, 1.0)
    o_ref[...] = summed / cnt[:, None]

  return pl.pallas_call(
      body,
      out_shape=jax.ShapeDtypeStruct((N, D), jnp.float32),
  )(partials, counts)


@jax.jit
def kernel(x, edge_index):
  src = edge_index[0].astype(jnp.int32).reshape(NW, CHUNKS, BATCH)
  dst = edge_index[1].astype(jnp.int32)
  dst_rows = dst.reshape(NW, CHUNKS, BATCH)
  dst_flat = dst.reshape(NW, 1, EPW)
  zeros = jnp.zeros((NP, D), jnp.float32)
  x = x.astype(jnp.float32)
  counts = _sc_counts(dst_flat)
  partials = _sc_accumulate(x, src, dst_rows, zeros)
  return _tc_combine(partials, counts)


# speedup vs baseline: 14.2715x; 1.0115x over previous
import functools

import jax
import jax.numpy as jnp
from jax import lax
from jax.experimental import pallas as pl
from jax.experimental.pallas import tpu as pltpu
from jax.experimental.pallas import tpu_sc as plsc

N = 10000
E = 320000
D = 128

NC = 2
NS = 16
NW = NC * NS
EPW = E // NW
BATCH = 80
CHUNKS = EPW // BATCH
NBUF = 3
STAGES = 5
NP = 10112
RPT = NP // NS

_SC_PARAMS = pltpu.CompilerParams(needs_layout_passes=False)


def _sc_counts(dst_flat):
  mesh = plsc.VectorSubcoreMesh(core_axis_name="c", subcore_axis_name="s")

  @functools.partial(
      pl.kernel,
      out_type=jax.ShapeDtypeStruct((NW, 1, N), jnp.float32),
      mesh=mesh,
      compiler_params=_SC_PARAMS,
      scratch_types=[
          pltpu.VMEM((EPW,), jnp.int32),
          pltpu.VMEM((N,), jnp.float32),
      ],
  )
  def body(dstf_hbm, cnt_hbm, dstf_v, cnt_v):
    c = lax.axis_index("c").astype(jnp.int32)
    s = lax.axis_index("s").astype(jnp.int32)
    wid = s * jnp.int32(NC) + c

    pltpu.sync_copy(dstf_hbm.at[wid, jnp.int32(0)], dstf_v)

    zeros16 = jnp.zeros((16,), jnp.float32)
    ones16 = jnp.ones((16,), jnp.float32)

    @pl.loop(jnp.int32(0), jnp.int32(N // 16))
    def _zero(i):
      cnt_v[pl.ds(i * jnp.int32(16), 16)] = zeros16

    @pl.loop(jnp.int32(0), jnp.int32(EPW // 16))
    def _count(i):
      idx = dstf_v[pl.ds(i * jnp.int32(16), 16)]
      plsc.addupdate_scatter(cnt_v, [idx], ones16)

    pltpu.sync_copy(cnt_v, cnt_hbm.at[wid, jnp.int32(0)])

  return body(dst_flat)


HC = CHUNKS // STAGES


def _sc_accumulate(x, src_idx, dst_idx, zeros):
  mesh = plsc.VectorSubcoreMesh(core_axis_name="c", subcore_axis_name="s")

  @functools.partial(
      pl.kernel,
      out_type=jax.ShapeDtypeStruct((NC, NP, D), jnp.float32),
      mesh=mesh,
      compiler_params=_SC_PARAMS,
      scratch_types=[
          pltpu.VMEM((HC, BATCH), jnp.int32),
          pltpu.VMEM((HC, BATCH), jnp.int32),
          pltpu.VMEM((NBUF, BATCH, D), jnp.float32),
          pltpu.VMEM_SHARED((NP, D), jnp.float32),
          pltpu.SemaphoreType.DMA((NBUF,)),
          pltpu.SemaphoreType.DMA((NBUF,)),
      ],
  )
  def body(x_hbm, srci_hbm, dsti_hbm, zeros_hbm, part_hbm,
           srci_v, dsti_v, rows_v, acc_sh, gsem, ssem):
    c = lax.axis_index("c").astype(jnp.int32)
    s = lax.axis_index("s").astype(jnp.int32)
    wid = s * jnp.int32(NC) + c

    row0 = pl.multiple_of(s * jnp.int32(RPT), RPT)
    pltpu.sync_copy(zeros_hbm.at[pl.ds(row0, RPT)],
                    acc_sh.at[pl.ds(row0, RPT)])

    plsc.subcore_barrier()

    def gather_start(g, b):
      pltpu.async_copy(x_hbm.at[srci_v.at[g]], rows_v.at[b], gsem.at[b])

    def gather_wait(g, b):
      pltpu.make_async_copy(
          x_hbm.at[srci_v.at[g]], rows_v.at[b], gsem.at[b]).wait()

    def scatter_start(g, b):
      pltpu.async_copy(rows_v.at[b], acc_sh.at[dsti_v.at[g]], ssem.at[b],
                       add=True)

    def scatter_wait(g, b):
      pltpu.make_async_copy(
          rows_v.at[b], acc_sh.at[dsti_v.at[g]], ssem.at[b]).wait()

    for st in range(STAGES):
      pltpu.sync_copy(srci_hbm.at[wid, jnp.int32(st)], srci_v)
      pltpu.sync_copy(dsti_hbm.at[wid, jnp.int32(st)], dsti_v)

      gather_start(jnp.int32(0), jnp.int32(0))

      @pl.loop(jnp.int32(0), jnp.int32(HC))
      def _chunk(g):
        b = lax.rem(g, jnp.int32(NBUF))

        @pl.when(g >= jnp.int32(NBUF - 1))
        def _():
          gp = g - jnp.int32(NBUF - 1)
          scatter_wait(gp, lax.rem(gp, jnp.int32(NBUF)))

        @pl.when(g + jnp.int32(1) < jnp.int32(HC))
        def _():
          gather_start(g + jnp.int32(1),
                       lax.rem(g + jnp.int32(1), jnp.int32(NBUF)))

        gather_wait(g, b)
        scatter_start(g, b)

      for p in range(NBUF - 1, 0, -1):
        scatter_wait(jnp.int32(HC - p), jnp.int32((HC - p) % NBUF))

    plsc.subcore_barrier()

    pltpu.sync_copy(acc_sh.at[pl.ds(row0, RPT)],
                    part_hbm.at[c].at[pl.ds(row0, RPT)])

  return body(x, src_idx, dst_idx, zeros)


def _tc_combine(partials, counts):
  def body(p_ref, c_ref, o_ref):
    summed = p_ref[0, :N, :] + p_ref[1, :N, :]
    cnt = jnp.sum(c_ref[:, 0, :], axis=0)
    cnt = jnp.maximum(cnt, 1.0)
    o_ref[...] = summed / cnt[:, None]

  return pl.pallas_call(
      body,
      out_shape=jax.ShapeDtypeStruct((N, D), jnp.float32),
  )(partials, counts)


@jax.jit
def kernel(x, edge_index):
  src = edge_index[0].astype(jnp.int32).reshape(NW, STAGES, HC, BATCH)
  dst = edge_index[1].astype(jnp.int32)
  dst_rows = dst.reshape(NW, STAGES, HC, BATCH)
  dst_flat = dst.reshape(NW, 1, EPW)
  zeros = jnp.zeros((NP, D), jnp.float32)
  x = x.astype(jnp.float32)
  counts = _sc_counts(dst_flat)
  partials = _sc_accumulate(x, src, dst_rows, zeros)
  return _tc_combine(partials, counts)
